# baseline (speedup 1.0000x reference)
import functools

import jax
import jax.numpy as jnp
from jax import lax
from jax.experimental import pallas as pl
from jax.experimental.pallas import tpu as pltpu
from jax.experimental.pallas import tpu_sc as plsc

B, L1, L2, D = 8, 512, 2048, 3
L2T = L2 // 128
SPLIT_T = 4
REM_T = L2T - SPLIT_T
NC, NS = 2, 16
NW = NC * NS
QPB = NW // B
QT = SPLIT_T // QPB
L2Q = QT * 128
T1 = 8
RC = T1 * 8
NCH = L1 // RC
LANES = 16
NG = L2Q // LANES
G = 4
TBLK = 3


def _ins3(m1, m2, m3, x):
    t1 = jnp.minimum(m1, x)
    r1 = jnp.maximum(m1, x)
    t2 = jnp.minimum(m2, r1)
    r2 = jnp.maximum(m2, r1)
    t3 = jnp.minimum(m3, r2)
    return t1, t2, t3


def _merge3(a, b):
    a1, a2, a3 = a
    b1, b2, b3 = b
    c1 = jnp.minimum(a1, b1)
    c2 = jnp.minimum(jnp.maximum(a1, b1), jnp.minimum(a2, b2))
    c3 = jnp.minimum(
        jnp.minimum(jnp.maximum(a1, b2), jnp.maximum(a2, b1)),
        jnp.minimum(a3, b3))
    return c1, c2, c3


def _sc_body(x_hbm, wb_hbm, out_hbm, buf0, buf1, m1, m2, m3, oub, wv,
             sem0, sem1, semw):
    cid = lax.axis_index("c")
    sid = lax.axis_index("s")
    wid = sid * NC + cid
    b_idx = wid // QPB
    q = wid % QPB

    pltpu.async_copy(wb_hbm, wv, semw).wait()

    inf = jnp.full((LANES,), jnp.inf, jnp.float32)

    def init_j(j, carry):
        for d in range(D):
            m1[d, pl.ds(j * LANES, LANES)] = inf
            m2[d, pl.ds(j * LANES, LANES)] = inf
            m3[d, pl.ds(j * LANES, LANES)] = inf
        return carry

    lax.fori_loop(0, NG, init_j, 0)

    bufs = (buf0, buf1)
    sems = (sem0, sem1)
    steps = [(d, c) for d in range(D) for c in range(NCH)]

    def start(i):
        d, c = steps[i]
        return pltpu.async_copy(
            x_hbm.at[b_idx, d, pl.ds(T1 * c, T1), pl.ds(QT * q, QT), :, :],
            bufs[i % 2], sems[i % 2])

    pending = start(0)
    for i, (d, c) in enumerate(steps):
        pending.wait()
        nxt = start(i + 1) if i + 1 < len(steps) else None
        buf = bufs[i % 2]

        def jj_body(jj, carry, buf=buf, d=d):
            st = []
            for gi in range(G):
                goff = (jj * G + gi) * LANES
                st += [m1[d, pl.ds(goff, LANES)],
                       m2[d, pl.ds(goff, LANES)],
                       m3[d, pl.ds(goff, LANES)]]

            def r_body(r, st):
                st = list(st)
                t = r >> 3
                ri = r & 7
                for gi in range(G):
                    g = jj * G + gi
                    tc = g >> 3
                    si = g & 7
                    x = buf[t, tc, ri, pl.ds(si * LANES, LANES)]
                    a, b2, c2 = _ins3(st[3 * gi], st[3 * gi + 1],
                                      st[3 * gi + 2], x)
                    st[3 * gi], st[3 * gi + 1], st[3 * gi + 2] = a, b2, c2
                return tuple(st)

            st = lax.fori_loop(0, RC, r_body, tuple(st), unroll=4)
            for gi in range(G):
                goff = (jj * G + gi) * LANES
                m1[d, pl.ds(goff, LANES)] = st[3 * gi]
                m2[d, pl.ds(goff, LANES)] = st[3 * gi + 1]
                m3[d, pl.ds(goff, LANES)] = st[3 * gi + 2]
            return carry

        lax.fori_loop(0, NG // G, jj_body, 0)
        pending = nxt

    w_lo = wv[pl.ds(0, LANES)]
    w_hi = wv[pl.ds(LANES, LANES)]

    def _w(i):
        return w_lo[i] if i < LANES else w_hi[i - LANES]

    ms = (m1, m2, m3)

    def blk_body(g, carry):
        goff = g * LANES
        tc = g >> 3
        si = g & 7
        vals = [[ms[k][d, pl.ds(goff, LANES)] for d in range(D)]
                for k in range(3)]
        for co in range(D):
            acc = jnp.zeros((LANES,), jnp.float32) + _w(27 + co)
            for k in range(3):
                for dd in range(D):
                    acc = acc + vals[k][dd] * _w((k * D + dd) * D + co)
            oub[co, tc, 0, pl.ds(si * LANES, LANES)] = acc
        return carry

    lax.fori_loop(0, NG, blk_body, 0)
    pltpu.sync_copy(
        oub, out_hbm.at[:, pl.ds(QT * q, QT), pl.ds(b_idx, 1), :])


def _tc_body(wb_ref, x_ref, o_ref):
    inf8 = jnp.full((8, 128), jnp.inf, jnp.float32)
    bsel = lax.broadcasted_iota(jnp.int32, (8, 128), 0)
    PG = 4
    pairs = [(b, d) for b in range(B) for d in range(D)]
    for j in range(TBLK):
        vals = {}
        for p0 in range(0, len(pairs), PG):
            grp = pairs[p0:p0 + PG]

            def rb(t, st, grp=grp, j=j):
                st = list(st)
                for i, (b, d) in enumerate(grp):
                    x = x_ref[b, d, t, j]
                    st[3 * i], st[3 * i + 1], st[3 * i + 2] = _ins3(
                        st[3 * i], st[3 * i + 1], st[3 * i + 2], x)
                return tuple(st)

            st = lax.fori_loop(0, L1 // 8, rb, (inf8,) * (3 * len(grp)),
                               unroll=2)
            for i, (b, d) in enumerate(grp):
                m = (st[3 * i], st[3 * i + 1], st[3 * i + 2])
                for sh in (4, 2, 1):
                    m = _merge3(m, tuple(pltpu.roll(mm, sh, 0) for mm in m))
                for k in range(3):
                    vals[(b, k, d)] = m[k]
        for co in range(D):
            acc_co = jnp.zeros((8, 128), jnp.float32)
            for b in range(B):
                acc = jnp.zeros((8, 128), jnp.float32) + wb_ref[27 + co]
                for k in range(3):
                    for dd in range(D):
                        acc = acc + vals[(b, k, dd)] * wb_ref[(k * D + dd) * D + co]
                acc_co = jnp.where(bsel == b, acc, acc_co)
            o_ref[co, j] = acc_co


@functools.partial(jax.jit)
def kernel(local_decisions, W, b):
    x = (local_decisions
         .transpose(0, 3, 1, 2)
         .reshape(B, D, L1 // 8, 8, L2T, 128)
         .transpose(0, 1, 2, 4, 3, 5))
    wb = jnp.concatenate(
        [W.reshape(-1), b, jnp.zeros((2,), jnp.float32)]).astype(jnp.float32)

    mesh = plsc.VectorSubcoreMesh(
        core_axis_name="c", subcore_axis_name="s",
        num_cores=NC, num_subcores=NS)
    out_sc = pl.kernel(
        _sc_body,
        out_type=jax.ShapeDtypeStruct((D, SPLIT_T, B, 128), jnp.float32),
        mesh=mesh,
        compiler_params=pltpu.CompilerParams(needs_layout_passes=False),
        scratch_types=[
            pltpu.VMEM((T1, QT, 8, 128), jnp.float32),
            pltpu.VMEM((T1, QT, 8, 128), jnp.float32),
            pltpu.VMEM((D, L2Q), jnp.float32),
            pltpu.VMEM((D, L2Q), jnp.float32),
            pltpu.VMEM((D, L2Q), jnp.float32),
            pltpu.VMEM((D, QT, 1, 128), jnp.float32),
            pltpu.VMEM((32,), jnp.float32),
            pltpu.SemaphoreType.DMA,
            pltpu.SemaphoreType.DMA,
            pltpu.SemaphoreType.DMA,
        ],
    )(x, wb)

    out_tc = pl.pallas_call(
        _tc_body,
        grid=(REM_T // TBLK,),
        in_specs=[
            pl.BlockSpec(memory_space=pltpu.SMEM),
            pl.BlockSpec(
                (B, D, L1 // 8, TBLK, 8, 128),
                lambda jj: (0, 0, 0, SPLIT_T // TBLK + jj, 0, 0)),
        ],
        out_specs=pl.BlockSpec(
            (D, TBLK, B, 128), lambda jj: (0, jj, 0, 0)),
        out_shape=jax.ShapeDtypeStruct((D, REM_T, B, 128), jnp.float32),
    )(wb, x)

    out = jnp.concatenate([out_sc, out_tc], axis=1)
    return out.transpose(2, 1, 3, 0).reshape(B, L2, D)


# speedup vs baseline: 1.0199x; 1.0199x over previous
import functools

import jax
import jax.numpy as jnp
from jax import lax
from jax.experimental import pallas as pl
from jax.experimental.pallas import tpu as pltpu
from jax.experimental.pallas import tpu_sc as plsc

B, L1, L2, D = 8, 512, 2048, 3
L2T = L2 // 128
SPLIT_T = 4
REM_T = L2T - SPLIT_T
NC, NS = 2, 16
NW = NC * NS
QPB = NW // B
QT = SPLIT_T // QPB
L2Q = QT * 128
T1 = 8
RC = T1 * 8
NCH = L1 // RC
LANES = 16
NG = L2Q // LANES
G = 4
TBLK = 2


def _ins3(m1, m2, m3, x):
    t1 = jnp.minimum(m1, x)
    r1 = jnp.maximum(m1, x)
    t2 = jnp.minimum(m2, r1)
    r2 = jnp.maximum(m2, r1)
    t3 = jnp.minimum(m3, r2)
    return t1, t2, t3


def _merge3(a, b):
    a1, a2, a3 = a
    b1, b2, b3 = b
    c1 = jnp.minimum(a1, b1)
    c2 = jnp.minimum(jnp.maximum(a1, b1), jnp.minimum(a2, b2))
    c3 = jnp.minimum(
        jnp.minimum(jnp.maximum(a1, b2), jnp.maximum(a2, b1)),
        jnp.minimum(a3, b3))
    return c1, c2, c3


def _sc_body(x_hbm, wb_hbm, out_hbm, buf0, buf1, m1, m2, m3, oub, wv,
             sem0, sem1, semw):
    cid = lax.axis_index("c")
    sid = lax.axis_index("s")
    wid = sid * NC + cid
    b_idx = wid // QPB
    q = wid % QPB

    pltpu.async_copy(wb_hbm, wv, semw).wait()

    inf = jnp.full((LANES,), jnp.inf, jnp.float32)

    def init_j(j, carry):
        for d in range(D):
            m1[d, pl.ds(j * LANES, LANES)] = inf
            m2[d, pl.ds(j * LANES, LANES)] = inf
            m3[d, pl.ds(j * LANES, LANES)] = inf
        return carry

    lax.fori_loop(0, NG, init_j, 0)

    bufs = (buf0, buf1)
    sems = (sem0, sem1)
    steps = [(d, c) for d in range(D) for c in range(NCH)]

    def start(i):
        d, c = steps[i]
        return pltpu.async_copy(
            x_hbm.at[b_idx, d, pl.ds(T1 * c, T1), pl.ds(QT * q, QT), :, :],
            bufs[i % 2], sems[i % 2])

    pending = start(0)
    for i, (d, c) in enumerate(steps):
        pending.wait()
        nxt = start(i + 1) if i + 1 < len(steps) else None
        buf = bufs[i % 2]

        def jj_body(jj, carry, buf=buf, d=d):
            st = []
            for gi in range(G):
                goff = (jj * G + gi) * LANES
                st += [m1[d, pl.ds(goff, LANES)],
                       m2[d, pl.ds(goff, LANES)],
                       m3[d, pl.ds(goff, LANES)]]

            def r_body(r, st):
                st = list(st)
                t = r >> 3
                ri = r & 7
                for gi in range(G):
                    g = jj * G + gi
                    tc = g >> 3
                    si = g & 7
                    x = buf[t, tc, ri, pl.ds(si * LANES, LANES)]
                    a, b2, c2 = _ins3(st[3 * gi], st[3 * gi + 1],
                                      st[3 * gi + 2], x)
                    st[3 * gi], st[3 * gi + 1], st[3 * gi + 2] = a, b2, c2
                return tuple(st)

            st = lax.fori_loop(0, RC, r_body, tuple(st), unroll=4)
            for gi in range(G):
                goff = (jj * G + gi) * LANES
                m1[d, pl.ds(goff, LANES)] = st[3 * gi]
                m2[d, pl.ds(goff, LANES)] = st[3 * gi + 1]
                m3[d, pl.ds(goff, LANES)] = st[3 * gi + 2]
            return carry

        lax.fori_loop(0, NG // G, jj_body, 0)
        pending = nxt

    w_lo = wv[pl.ds(0, LANES)]
    w_hi = wv[pl.ds(LANES, LANES)]

    def _w(i):
        return w_lo[i] if i < LANES else w_hi[i - LANES]

    ms = (m1, m2, m3)

    def blk_body(g, carry):
        goff = g * LANES
        tc = g >> 3
        si = g & 7
        vals = [[ms[k][d, pl.ds(goff, LANES)] for d in range(D)]
                for k in range(3)]
        for co in range(D):
            acc = jnp.zeros((LANES,), jnp.float32) + _w(27 + co)
            for k in range(3):
                for dd in range(D):
                    acc = acc + vals[k][dd] * _w((k * D + dd) * D + co)
            oub[co, tc, 0, pl.ds(si * LANES, LANES)] = acc
        return carry

    lax.fori_loop(0, NG, blk_body, 0)
    pltpu.sync_copy(
        oub, out_hbm.at[:, pl.ds(QT * q, QT), pl.ds(b_idx, 1), :])


def _tc_body(wb_ref, x_ref, o_ref):
    inf8 = jnp.full((8, 128), jnp.inf, jnp.float32)
    bsel = lax.broadcasted_iota(jnp.int32, (8, 128), 0)
    PG = 4
    pairs = [(b, d) for b in range(B) for d in range(D)]
    for j in range(TBLK):
        vals = {}
        for p0 in range(0, len(pairs), PG):
            grp = pairs[p0:p0 + PG]

            def rb(t, st, grp=grp, j=j):
                st = list(st)
                for i, (b, d) in enumerate(grp):
                    x = x_ref[b, d, t, j]
                    st[3 * i], st[3 * i + 1], st[3 * i + 2] = _ins3(
                        st[3 * i], st[3 * i + 1], st[3 * i + 2], x)
                return tuple(st)

            st = lax.fori_loop(0, L1 // 8, rb, (inf8,) * (3 * len(grp)),
                               unroll=2)
            for i, (b, d) in enumerate(grp):
                m = (st[3 * i], st[3 * i + 1], st[3 * i + 2])
                for sh in (4, 2, 1):
                    m = _merge3(m, tuple(pltpu.roll(mm, sh, 0) for mm in m))
                for k in range(3):
                    vals[(b, k, d)] = m[k]
        for co in range(D):
            acc_co = jnp.zeros((8, 128), jnp.float32)
            for b in range(B):
                acc = jnp.zeros((8, 128), jnp.float32) + wb_ref[27 + co]
                for k in range(3):
                    for dd in range(D):
                        acc = acc + vals[(b, k, dd)] * wb_ref[(k * D + dd) * D + co]
                acc_co = jnp.where(bsel == b, acc, acc_co)
            o_ref[co, j] = acc_co


@functools.partial(jax.jit)
def kernel(local_decisions, W, b):
    x = (local_decisions
         .transpose(0, 3, 1, 2)
         .reshape(B, D, L1 // 8, 8, L2T, 128)
         .transpose(0, 1, 2, 4, 3, 5))
    wb = jnp.concatenate(
        [W.reshape(-1), b, jnp.zeros((2,), jnp.float32)]).astype(jnp.float32)

    mesh = plsc.VectorSubcoreMesh(
        core_axis_name="c", subcore_axis_name="s",
        num_cores=NC, num_subcores=NS)
    out_sc = pl.kernel(
        _sc_body,
        out_type=jax.ShapeDtypeStruct((D, SPLIT_T, B, 128), jnp.float32),
        mesh=mesh,
        compiler_params=pltpu.CompilerParams(needs_layout_passes=False),
        scratch_types=[
            pltpu.VMEM((T1, QT, 8, 128), jnp.float32),
            pltpu.VMEM((T1, QT, 8, 128), jnp.float32),
            pltpu.VMEM((D, L2Q), jnp.float32),
            pltpu.VMEM((D, L2Q), jnp.float32),
            pltpu.VMEM((D, L2Q), jnp.float32),
            pltpu.VMEM((D, QT, 1, 128), jnp.float32),
            pltpu.VMEM((32,), jnp.float32),
            pltpu.SemaphoreType.DMA,
            pltpu.SemaphoreType.DMA,
            pltpu.SemaphoreType.DMA,
        ],
    )(x, wb)

    out_tc = pl.pallas_call(
        _tc_body,
        grid=(REM_T // TBLK,),
        in_specs=[
            pl.BlockSpec(memory_space=pltpu.SMEM),
            pl.BlockSpec(
                (B, D, L1 // 8, TBLK, 8, 128),
                lambda jj: (0, 0, 0, SPLIT_T // TBLK + jj, 0, 0)),
        ],
        out_specs=pl.BlockSpec(
            (D, TBLK, B, 128), lambda jj: (0, jj, 0, 0)),
        out_shape=jax.ShapeDtypeStruct((D, REM_T, B, 128), jnp.float32),
    )(wb, x)

    out = jnp.concatenate([out_sc, out_tc], axis=1)
    return out.transpose(2, 1, 3, 0).reshape(B, L2, D)
